# baseline (speedup 1.0000x reference)
import functools

import jax
import jax.numpy as jnp
from jax.experimental import pallas as pl
from jax.experimental.pallas import tpu as pltpu


def _scse_body(x_ref, w1s_ref, b1_ref, w2_ref, b2_ref, ws_ref, bs_ref,
               o_ref, stash_ref, acc_ref, sseacc_ref,
               cg0_ref, cg1_ref, sg0_ref, sg1_ref,
               *, c, c_tile, grp):
    n = pl.program_id(1)
    t = pl.program_id(2)
    nt = pl.num_programs(2)
    parity = jax.lax.rem(n, 2)

    @pl.when(n < grp)
    def _():
        x = x_ref[0]
        stash_ref[pl.ds(parity * c + t * c_tile, c_tile), :] = x

        acc_ref[pl.ds(t * c_tile, c_tile), :] = jnp.dot(
            x, jnp.ones((x.shape[1], 1), jnp.float32),
            preferred_element_type=jnp.float32)

        wpart = jnp.dot(ws_ref[:, pl.ds(t * c_tile, c_tile)], x,
                        preferred_element_type=jnp.float32)
        sse = jnp.where(t == 0, wpart, sseacc_ref[...] + wpart)
        sseacc_ref[...] = sse

        @pl.when(t == nt - 1)
        def _():
            hidden = jnp.dot(w1s_ref[...], acc_ref[...],
                             preferred_element_type=jnp.float32) + b1_ref[...]
            hidden = jnp.maximum(hidden, 0.0)
            logit = jnp.dot(w2_ref[...], hidden,
                            preferred_element_type=jnp.float32) + b2_ref[...]
            cg = jax.nn.sigmoid(logit)
            sg = jax.nn.sigmoid(sse + bs_ref[...])

            @pl.when(parity == 0)
            def _():
                cg0_ref[...] = cg
                sg0_ref[...] = sg

            @pl.when(parity == 1)
            def _():
                cg1_ref[...] = cg
                sg1_ref[...] = sg

    @pl.when(n > 0)
    def _():
        prev = jax.lax.rem(n + 1, 2)
        xp = stash_ref[pl.ds(prev * c + t * c_tile, c_tile), :]
        csel = jnp.where(
            prev == 0,
            cg0_ref[pl.ds(t * c_tile, c_tile), :],
            cg1_ref[pl.ds(t * c_tile, c_tile), :])
        ssel = jnp.where(prev == 0, sg0_ref[...], sg1_ref[...])
        o_ref[0] = xp * (csel + ssel)


def kernel(x_nchw, w1, b1, w2, b2, ws, bs):
    N, C, H, W = x_nchw.shape
    HW = H * W
    Cr = w1.shape[0]
    x = x_nchw.reshape(N, C, HW)

    nsplit = 2 if N % 2 == 0 else 1
    grp = N // nsplit
    c_tile = 256 if C % 256 == 0 else C
    nt = C // c_tile

    w1s = (w1.reshape(Cr, C) * (1.0 / float(HW))).astype(jnp.float32)
    b1c = b1.reshape(Cr, 1).astype(jnp.float32)
    w2m = w2.reshape(C, Cr).astype(jnp.float32)
    b2c = b2.reshape(C, 1).astype(jnp.float32)
    wsr = ws.reshape(1, C).astype(jnp.float32)
    bss = bs.reshape(1, 1).astype(jnp.float32)

    def x_map(s, n, t):
        samp = s * grp + jnp.minimum(n, grp - 1)
        ct = jnp.where(n == grp, nt - 1, t)
        return (samp, ct, 0)

    def o_map(s, n, t):
        samp = s * grp + jnp.maximum(n - 1, 0)
        ct = jnp.where(n == 0, 0, t)
        return (samp, ct, 0)

    def whole(a):
        return pl.BlockSpec(a.shape, lambda s, n, t: (0,) * a.ndim)

    out = pl.pallas_call(
        functools.partial(_scse_body, c=C, c_tile=c_tile, grp=grp),
        out_shape=jax.ShapeDtypeStruct((N, C, HW), x.dtype),
        grid=(nsplit, grp + 1, nt),
        in_specs=[pl.BlockSpec((1, c_tile, HW), x_map),
                  whole(w1s), whole(b1c), whole(w2m), whole(b2c),
                  whole(wsr), whole(bss)],
        out_specs=pl.BlockSpec((1, c_tile, HW), o_map),
        scratch_shapes=[pltpu.VMEM((2 * C, HW), jnp.float32),
                        pltpu.VMEM((C, 1), jnp.float32),
                        pltpu.VMEM((1, HW), jnp.float32),
                        pltpu.VMEM((C, 1), jnp.float32),
                        pltpu.VMEM((C, 1), jnp.float32),
                        pltpu.VMEM((1, HW), jnp.float32),
                        pltpu.VMEM((1, HW), jnp.float32)],
        compiler_params=pltpu.CompilerParams(
            dimension_semantics=("parallel", "arbitrary", "arbitrary"),
            vmem_limit_bytes=96 << 20),
    )(x, w1s, b1c, w2m, b2c, wsr, bss)
    return out.reshape(N, C, H, W)


# speedup vs baseline: 1.0633x; 1.0633x over previous
import functools

import jax
import jax.numpy as jnp
from jax.experimental import pallas as pl
from jax.experimental.pallas import tpu as pltpu


def _scse_body(x_ref, w1s_ref, b1_ref, w2_ref, b2_ref, ws_ref, bs_ref, o_ref):
    x = x_ref[0]
    c, hw = x.shape

    colsum = jnp.sum(x, axis=1, keepdims=True)
    pix_logit = jnp.dot(ws_ref[...], x,
                        precision=jax.lax.Precision.DEFAULT,
                        preferred_element_type=jnp.float32) + bs_ref[...]
    pix_gate = jax.nn.sigmoid(pix_logit)

    hidden = jnp.dot(w1s_ref[...], colsum,
                     preferred_element_type=jnp.float32) + b1_ref[...]
    hidden = jnp.maximum(hidden, 0.0)
    chan_logit = jnp.dot(w2_ref[...], hidden,
                         preferred_element_type=jnp.float32) + b2_ref[...]
    chan_gate = jax.nn.sigmoid(chan_logit)

    o_ref[0] = x * (chan_gate + pix_gate)


def kernel(x_nchw, w1, b1, w2, b2, ws, bs):
    N, C, H, W = x_nchw.shape
    HW = H * W
    Cr = w1.shape[0]
    x = x_nchw.reshape(N, C, HW)

    w1s = (w1.reshape(Cr, C) * (1.0 / float(HW))).astype(jnp.float32)
    b1c = b1.reshape(Cr, 1).astype(jnp.float32)
    w2m = w2.reshape(C, Cr).astype(jnp.float32)
    b2c = b2.reshape(C, 1).astype(jnp.float32)
    wsr = ws.reshape(1, C).astype(jnp.float32)
    bss = bs.reshape(1, 1).astype(jnp.float32)

    sample_spec = pl.BlockSpec((1, C, HW), lambda n: (n, 0, 0))

    def whole(a):
        return pl.BlockSpec(a.shape, lambda n: (0,) * a.ndim)

    out = pl.pallas_call(
        _scse_body,
        out_shape=jax.ShapeDtypeStruct((N, C, HW), x.dtype),
        grid=(N,),
        in_specs=[sample_spec,
                  whole(w1s), whole(b1c), whole(w2m), whole(b2c),
                  whole(wsr), whole(bss)],
        out_specs=sample_spec,
        compiler_params=pltpu.CompilerParams(
            dimension_semantics=("parallel",),
            vmem_limit_bytes=96 << 20),
    )(x, w1s, b1c, w2m, b2c, wsr, bss)
    return out.reshape(N, C, H, W)
